# Initial kernel scaffold; baseline (speedup 1.0000x reference)
import jax
import jax.numpy as jnp
from jax.experimental import pallas as pl


def kernel(x, tables):
    raise NotImplementedError("write your pallas kernel here")


# speedup vs baseline: 1.2003x; 1.2003x over previous
import functools

import jax
import jax.numpy as jnp
from jax import lax
from jax.experimental import pallas as pl
from jax.experimental.pallas import tpu as pltpu
from jax.experimental.pallas import tpu_sc as plsc

LANES = 16


@functools.lru_cache(maxsize=None)
def _make_sc_kernel(T, D, C, chunk):
    info = plsc.get_sparse_core_info()
    NC, NS = info.num_cores, info.num_subcores
    NW = NC * NS
    tpw = T // NW
    n_chunks = tpw // chunk
    mesh = plsc.VectorSubcoreMesh(core_axis_name="c", subcore_axis_name="s")

    @functools.partial(
        pl.kernel,
        mesh=mesh,
        out_type=jax.ShapeDtypeStruct((T, D), jnp.float32),
        scratch_types=[
            pltpu.VMEM((chunk * C,), jnp.int32),
            pltpu.VMEM((chunk * C, D), jnp.float32),
            pltpu.VMEM((chunk, D), jnp.float32),
            pltpu.SemaphoreType.DMA,
        ],
    )
    def k(idx_hbm, ftab_hbm, out_hbm, idx_v, rows_v, out_v, sem):
        wid = lax.axis_index("s") * NC + lax.axis_index("c")
        base = wid * tpw

        def chunk_body(i, carry):
            tok0 = base + i * chunk
            pltpu.sync_copy(idx_hbm.at[pl.ds(tok0 * C, chunk * C)], idx_v)
            pltpu.async_copy(ftab_hbm.at[idx_v], rows_v, sem).wait()

            def slice_body(s, c2):
                off = s * LANES
                for j in range(chunk):
                    acc = rows_v[j * C, pl.ds(off, LANES)]
                    for c in range(1, C):
                        acc = acc + rows_v[j * C + c, pl.ds(off, LANES)]
                    out_v[j, pl.ds(off, LANES)] = acc
                return c2

            lax.fori_loop(0, D // LANES, slice_body, 0)
            pltpu.sync_copy(out_v, out_hbm.at[pl.ds(tok0, chunk)])
            return carry

        lax.fori_loop(0, n_chunks, chunk_body, 0)

    return k


def kernel(x, tables):
    B, C, L = x.shape
    _, V, D = tables.shape
    T = B * L
    idx = x.astype(jnp.int32).transpose(0, 2, 1) + (
        jnp.arange(C, dtype=jnp.int32) * V
    )
    idx = idx.reshape(T * C)
    ftab = tables.reshape(C * V, D)
    out = _make_sc_kernel(T, D, C, 4)(idx, ftab)
    return out.reshape(B, L, D)


# speedup vs baseline: 1.9918x; 1.6594x over previous
import functools

import jax
import jax.numpy as jnp
from jax import lax
from jax.experimental import pallas as pl
from jax.experimental.pallas import tpu as pltpu
from jax.experimental.pallas import tpu_sc as plsc

LANES = 16


@functools.lru_cache(maxsize=None)
def _make_sc_kernel(T, D, C, chunk):
    info = plsc.get_sparse_core_info()
    NC, NS = info.num_cores, info.num_subcores
    NW = NC * NS
    tpw = T // NW
    n_chunks = tpw // chunk
    n_groups = n_chunks // 2
    mesh = plsc.VectorSubcoreMesh(core_axis_name="c", subcore_axis_name="s")

    @functools.partial(
        pl.kernel,
        mesh=mesh,
        out_type=jax.ShapeDtypeStruct((T, D), jnp.float32),
        scratch_types=[
            pltpu.VMEM((tpw * C,), jnp.int32),
            pltpu.VMEM((chunk * C, D), jnp.float32),
            pltpu.VMEM((chunk * C, D), jnp.float32),
            pltpu.VMEM((chunk, D), jnp.float32),
            pltpu.SemaphoreType.DMA,
            pltpu.SemaphoreType.DMA,
        ],
    )
    def k(idx_hbm, ftab_hbm, out_hbm, idx_v, rows0, rows1, out_v, sem0, sem1):
        wid = lax.axis_index("s") * NC + lax.axis_index("c")
        base = wid * tpw
        pltpu.sync_copy(idx_hbm.at[pl.ds(base * C, tpw * C)], idx_v)

        def start_gather(ci, rows, sem):
            off = ci * (chunk * C)
            pltpu.async_copy(
                ftab_hbm.at[idx_v.at[pl.ds(off, chunk * C)]], rows, sem
            )

        def wait_gather(rows, sem):
            pltpu.make_async_copy(
                ftab_hbm.at[pl.ds(0, chunk * C)], rows, sem
            ).wait()

        def phase(ci, next_ci, rows, sem):
            wait_gather(rows, sem)

            def slice_body(s, c2):
                off = s * LANES
                for j in range(chunk):
                    acc = rows[j * C, pl.ds(off, LANES)]
                    for c in range(1, C):
                        acc = acc + rows[j * C + c, pl.ds(off, LANES)]
                    out_v[j, pl.ds(off, LANES)] = acc
                return c2

            lax.fori_loop(0, D // LANES, slice_body, 0)
            start_gather(next_ci, rows, sem)
            pltpu.sync_copy(out_v, out_hbm.at[pl.ds(base + ci * chunk, chunk)])

        start_gather(0, rows0, sem0)
        start_gather(1, rows1, sem1)

        def group(g, carry):
            c0 = 2 * g
            phase(c0, jnp.minimum(c0 + 2, n_chunks - 1), rows0, sem0)
            phase(c0 + 1, jnp.minimum(c0 + 3, n_chunks - 1), rows1, sem1)
            return carry

        lax.fori_loop(0, n_groups, group, 0)
        wait_gather(rows0, sem0)
        wait_gather(rows1, sem1)

    return k


def kernel(x, tables):
    B, C, L = x.shape
    _, V, D = tables.shape
    T = B * L
    idx = x.astype(jnp.int32).transpose(0, 2, 1) + (
        jnp.arange(C, dtype=jnp.int32) * V
    )
    idx = idx.reshape(T * C)
    ftab = tables.reshape(C * V, D)
    out = _make_sc_kernel(T, D, C, 4)(idx, ftab)
    return out.reshape(B, L, D)
